# baseline (speedup 1.0000x reference)
import jax
import jax.numpy as jnp
from jax import lax
from jax.experimental import pallas as pl
from jax.experimental.pallas import tpu as pltpu
from jax.experimental.pallas import tpu_sc as plsc

_BT = 1024
_NW = 32


def _dense_body(x_ref, wt_ref, probs_ref, meanp_ref, z_ref, ent_ref):
    i = pl.program_id(0)
    nsteps = pl.num_programs(0)
    bt = x_ref.shape[0]
    t_total = bt * nsteps

    logits = jnp.dot(x_ref[...], wt_ref[...],
                     preferred_element_type=jnp.float32)
    lt = logits.T

    m = jnp.max(lt, axis=0, keepdims=True)
    eu = jnp.exp(lt - m)
    s = jnp.sum(eu, axis=0, keepdims=True)
    probs = eu / s
    probs_ref[...] = probs

    lse = m + jnp.log(s)
    z_blk = jnp.sum(lse * lse)
    ent_blk = -jnp.sum(probs * jnp.log(probs + 1e-10))
    psum_blk = jnp.sum(probs, axis=1, keepdims=True)

    @pl.when(i == 0)
    def _init():
        meanp_ref[...] = jnp.zeros_like(meanp_ref)
        z_ref[0, 0] = 0.0
        ent_ref[0, 0] = 0.0

    meanp_ref[...] += psum_blk
    z_ref[0, 0] += z_blk
    ent_ref[0, 0] += ent_blk

    @pl.when(i == nsteps - 1)
    def _fin():
        meanp_ref[...] = meanp_ref[...] / t_total
        z_ref[0, 0] = z_ref[0, 0] / t_total
        ent_ref[0, 0] = ent_ref[0, 0] / t_total


def _fin_body(idx_ref, meanp_ref, counts_ref, lb_ref, bal_ref):
    i = pl.program_id(0)
    nsteps = pl.num_programs(0)
    k_top, bt = idx_ref.shape
    e = meanp_ref.shape[0]
    t_total = bt * nsteps

    iota_e = lax.broadcasted_iota(jnp.int32, (e, bt), 0)
    acc = jnp.zeros((e, bt), jnp.float32)
    for r in range(k_top):
        row = idx_ref[r, :].reshape(1, bt)
        acc = acc + (iota_e == row).astype(jnp.float32)
    counts_blk = jnp.sum(acc, axis=1, keepdims=True)

    @pl.when(i == 0)
    def _init():
        counts_ref[...] = jnp.zeros_like(counts_ref)
        lb_ref[0, 0] = 0.0
        bal_ref[0, 0] = 0.0

    counts_ref[...] += counts_blk

    @pl.when(i == nsteps - 1)
    def _fin():
        frac = counts_ref[...] / (t_total * k_top)
        lb_ref[0, 0] = e * jnp.sum(frac * meanp_ref[...])
        bal_ref[0, 0] = jnp.max(frac) * e


def _sc_topk(probs_hbm, idx_hbm, w_hbm, buf, idxbuf, wbuf):
    nc = 2
    e = probs_hbm.shape[0]
    t = probs_hbm.shape[1]
    chunk = t // _NW
    k_top = idx_hbm.shape[0]
    wid = lax.axis_index("s") * nc + lax.axis_index("c")
    base = wid * chunk

    pltpu.sync_copy(probs_hbm.at[:, pl.ds(base, chunk)], buf)

    neg1 = jnp.full((16,), -1.0, jnp.float32)
    zero_i = jnp.zeros((16,), jnp.int32)

    def group(g, carry):
        col = g * 16
        rv = [neg1] * k_top
        ri = [zero_i] * k_top
        for ex in range(e):
            v = buf[ex, pl.ds(col, 16)]
            vi = jnp.full((16,), ex, jnp.int32)
            for r in range(k_top):
                c = v > rv[r]
                nv = jnp.where(c, v, rv[r])
                ni = jnp.where(c, vi, ri[r])
                v = jnp.where(c, rv[r], v)
                vi = jnp.where(c, ri[r], vi)
                rv[r] = nv
                ri[r] = ni
        wsum = rv[0]
        for r in range(1, k_top):
            wsum = wsum + rv[r]
        for r in range(k_top):
            idxbuf[r, pl.ds(col, 16)] = ri[r]
            wbuf[r, pl.ds(col, 16)] = rv[r] / wsum
        return carry

    lax.fori_loop(0, chunk // 16, group, 0)

    pltpu.sync_copy(idxbuf, idx_hbm.at[:, pl.ds(base, chunk)])
    pltpu.sync_copy(wbuf, w_hbm.at[:, pl.ds(base, chunk)])


def kernel(hidden_states, W):
    b, s, h = hidden_states.shape
    e = W.shape[0]
    k_top = 8
    t = b * s
    bt = _BT
    chunk = t // _NW

    x2 = hidden_states.reshape(t, h)
    wt = W.T

    smem_scalar = pl.BlockSpec((1, 1), lambda i: (0, 0),
                               memory_space=pltpu.SMEM)
    probs, meanp, z, ent = pl.pallas_call(
        _dense_body,
        grid=(t // bt,),
        in_specs=(
            pl.BlockSpec((bt, h), lambda i: (i, 0)),
            pl.BlockSpec((h, e), lambda i: (0, 0)),
        ),
        out_specs=(
            pl.BlockSpec((e, bt), lambda i: (0, i)),
            pl.BlockSpec((e, 1), lambda i: (0, 0)),
            smem_scalar, smem_scalar,
        ),
        out_shape=(
            jax.ShapeDtypeStruct((e, t), jnp.float32),
            jax.ShapeDtypeStruct((e, 1), jnp.float32),
            jax.ShapeDtypeStruct((1, 1), jnp.float32),
            jax.ShapeDtypeStruct((1, 1), jnp.float32),
        ),
        compiler_params=pltpu.CompilerParams(
            dimension_semantics=("arbitrary",)),
    )(x2, wt)

    mesh = plsc.VectorSubcoreMesh(core_axis_name="c", subcore_axis_name="s")
    idx8, w8 = pl.kernel(
        _sc_topk,
        mesh=mesh,
        out_type=[
            jax.ShapeDtypeStruct((k_top, t), jnp.int32),
            jax.ShapeDtypeStruct((k_top, t), jnp.float32),
        ],
        scratch_types=[
            pltpu.VMEM((e, chunk), jnp.float32),
            pltpu.VMEM((k_top, chunk), jnp.int32),
            pltpu.VMEM((k_top, chunk), jnp.float32),
        ],
    )(probs)

    bt3 = 1024
    counts, lb, bal = pl.pallas_call(
        _fin_body,
        grid=(t // bt3,),
        in_specs=(
            pl.BlockSpec((k_top, bt3), lambda i: (0, i)),
            pl.BlockSpec((e, 1), lambda i: (0, 0)),
        ),
        out_specs=(
            pl.BlockSpec((e, 1), lambda i: (0, 0)),
            smem_scalar, smem_scalar,
        ),
        out_shape=(
            jax.ShapeDtypeStruct((e, 1), jnp.float32),
            jax.ShapeDtypeStruct((1, 1), jnp.float32),
            jax.ShapeDtypeStruct((1, 1), jnp.float32),
        ),
        compiler_params=pltpu.CompilerParams(
            dimension_semantics=("arbitrary",)),
    )(idx8, meanp)

    return (idx8.T.reshape(b, s, k_top), w8.T.reshape(b, s, k_top),
            lb[0, 0], z[0, 0], bal[0, 0], ent[0, 0],
            counts.reshape(e), meanp.reshape(e))


# speedup vs baseline: 1.5850x; 1.5850x over previous
import jax
import jax.numpy as jnp
from jax import lax
from jax.experimental import pallas as pl
from jax.experimental.pallas import tpu as pltpu

_BT = 1024


def _router_body(x_ref, wt_ref, idx_ref, w_ref, counts_ref, psum_ref,
                 lb_ref, z_ref, bal_ref, ent_ref):
    i = pl.program_id(0)
    nsteps = pl.num_programs(0)
    bt = x_ref.shape[0]
    e = wt_ref.shape[1]
    t_total = bt * nsteps
    k_top = idx_ref.shape[0]

    logits = jnp.dot(x_ref[...], wt_ref[...],
                     preferred_element_type=jnp.float32)
    lt = logits.T

    m = jnp.max(lt, axis=0, keepdims=True)
    eu = jnp.exp(lt - m)
    s = jnp.sum(eu, axis=0, keepdims=True)
    probs = eu / s

    lse = m + jnp.log(s)
    z_blk = jnp.sum(lse * lse)
    ent_blk = -jnp.sum(probs * jnp.log(probs + 1e-10))
    psum_blk = jnp.sum(probs, axis=1, keepdims=True)

    iota_e = lax.broadcasted_iota(jnp.int32, (e, bt), 0)
    p = probs
    cnt = jnp.zeros((e, bt), jnp.int32)
    idx_rows = []
    w_rows = []
    for _ in range(k_top):
        wk = jnp.max(p, axis=0, keepdims=True)
        idxk = jnp.min(jnp.where(p == wk, iota_e, e), axis=0,
                       keepdims=True)
        onehot = iota_e == idxk
        cnt = cnt + onehot.astype(jnp.int32)
        idx_rows.append(idxk)
        w_rows.append(wk)
        p = jnp.where(onehot, -1.0, p)

    w_all = jnp.concatenate(w_rows, axis=0)
    wsum = jnp.sum(w_all, axis=0, keepdims=True)
    idx_ref[...] = jnp.concatenate(idx_rows, axis=0)
    w_ref[...] = w_all / wsum

    counts_blk = jnp.sum(cnt.astype(jnp.float32), axis=1,
                         keepdims=True)

    @pl.when(i == 0)
    def _init():
        counts_ref[...] = jnp.zeros_like(counts_ref)
        psum_ref[...] = jnp.zeros_like(psum_ref)
        z_ref[0, 0] = 0.0
        ent_ref[0, 0] = 0.0
        lb_ref[0, 0] = 0.0
        bal_ref[0, 0] = 0.0

    counts_ref[...] += counts_blk
    psum_ref[...] += psum_blk
    z_ref[0, 0] += z_blk
    ent_ref[0, 0] += ent_blk

    @pl.when(i == nsteps - 1)
    def _finalize():
        counts_f = counts_ref[...]
        frac = counts_f / (t_total * k_top)
        meanp = psum_ref[...] / t_total
        psum_ref[...] = meanp
        lb_ref[0, 0] = e * jnp.sum(frac * meanp)
        bal_ref[0, 0] = jnp.max(frac) * e
        z_ref[0, 0] = z_ref[0, 0] / t_total
        ent_ref[0, 0] = ent_ref[0, 0] / t_total


def kernel(hidden_states, W):
    b, s, h = hidden_states.shape
    e = W.shape[0]
    k_top = 8
    t = b * s
    bt = _BT
    grid = (t // bt,)

    x2 = hidden_states.reshape(t, h)
    wt = W.T

    smem_scalar = pl.BlockSpec((1, 1), lambda i: (0, 0),
                               memory_space=pltpu.SMEM)
    out_shapes = (
        jax.ShapeDtypeStruct((k_top, t), jnp.int32),
        jax.ShapeDtypeStruct((k_top, t), jnp.float32),
        jax.ShapeDtypeStruct((e, 1), jnp.float32),
        jax.ShapeDtypeStruct((e, 1), jnp.float32),
        jax.ShapeDtypeStruct((1, 1), jnp.float32),
        jax.ShapeDtypeStruct((1, 1), jnp.float32),
        jax.ShapeDtypeStruct((1, 1), jnp.float32),
        jax.ShapeDtypeStruct((1, 1), jnp.float32),
    )
    out_specs = (
        pl.BlockSpec((k_top, bt), lambda i: (0, i)),
        pl.BlockSpec((k_top, bt), lambda i: (0, i)),
        pl.BlockSpec((e, 1), lambda i: (0, 0)),
        pl.BlockSpec((e, 1), lambda i: (0, 0)),
        smem_scalar, smem_scalar, smem_scalar, smem_scalar,
    )
    in_specs = (
        pl.BlockSpec((bt, h), lambda i: (i, 0)),
        pl.BlockSpec((h, e), lambda i: (0, 0)),
    )

    idx, w, counts, meanp, lb, z, bal, ent = pl.pallas_call(
        _router_body,
        grid=grid,
        in_specs=in_specs,
        out_specs=out_specs,
        out_shape=out_shapes,
        compiler_params=pltpu.CompilerParams(
            dimension_semantics=("arbitrary",)),
    )(x2, wt)

    return (idx.T.reshape(b, s, k_top), w.T.reshape(b, s, k_top),
            lb[0, 0], z[0, 0], bal[0, 0], ent[0, 0],
            counts.reshape(e), meanp.reshape(e))
